# baseline (speedup 1.0000x reference)
import functools

import jax
import jax.numpy as jnp
from jax import lax
from jax.experimental import pallas as pl
from jax.experimental.pallas import tpu as pltpu
from jax.experimental.pallas import tpu_sc as plsc

_M = 0.25
_A = 64.0 * 1.4426950408889634
_LN2 = 0.6931471805599453
_NEG_INF = float("-inf")

_TC_ROWS = 512
_NC = 2
_NS = 16
_NW = _NC * _NS
_CH = 20000
_U = 10


def _tc_loss_kernel(label_ref, x_ref, out_ref, m_ref, s_ref, g_ref, *,
                    n_cols, block_cols):
    k = pl.program_id(0)
    nk = pl.num_programs(0)

    @pl.when(k == 0)
    def _init():
        m_ref[...] = jnp.full(m_ref.shape, _NEG_INF, m_ref.dtype)
        s_ref[...] = jnp.zeros(s_ref.shape, s_ref.dtype)
        g_ref[...] = jnp.zeros(g_ref.shape, g_ref.dtype)

    def _accum(mask_invalid):
        x = x_ref[...]
        labloc = label_ref[...] - k * block_cols
        col = jax.lax.broadcasted_iota(jnp.int32, x.shape, 1)
        is_lab = col == labloc
        q = x * x * _A - (_A / 16.0)
        dense = jnp.where(x > -_M, q, 0.0)
        if mask_invalid:
            drop = is_lab | (col >= n_cols - k * block_cols)
        else:
            drop = is_lab
        l2 = jnp.where(drop, _NEG_INF, dense)
        g_ref[...] += jnp.sum(jnp.where(is_lab, x, 0.0), axis=1,
                              keepdims=True)
        bm = jnp.max(l2, axis=1, keepdims=True)
        m_old = m_ref[...]
        m_new = jnp.maximum(m_old, bm)
        s_ref[...] = s_ref[...] * jnp.exp2(m_old - m_new) + jnp.sum(
            jnp.exp2(l2 - m_new), axis=1, keepdims=True)
        m_ref[...] = m_new

    @pl.when(k < nk - 1)
    def _main():
        _accum(False)

    @pl.when(k == nk - 1)
    def _last():
        _accum(True)

        g = g_ref[...]
        spec2 = jnp.where(g < 1.0 + _M,
                          (2.0 * _A) * g - g * g * _A - 0.9375 * _A, 0.0)
        m2 = m_ref[...]
        s_true = s_ref[...] + jnp.exp2(spec2 - m2)
        loss = (m2 + jnp.log2(s_true) - spec2) * _LN2
        out_ref[0, 0] = jnp.sum(loss)


def _tc_partial(inp, lab2, block_cols=2048):
    b, c = inp.shape
    r = _TC_ROWS
    nk = pl.cdiv(c, block_cols)
    return pl.pallas_call(
        functools.partial(_tc_loss_kernel, n_cols=c, block_cols=block_cols),
        grid=(nk,),
        in_specs=[
            pl.BlockSpec((r, 1), lambda k: (0, 0)),
            pl.BlockSpec((r, block_cols), lambda k: (0, k)),
        ],
        out_specs=pl.BlockSpec(memory_space=pltpu.SMEM),
        out_shape=jax.ShapeDtypeStruct((1, 1), jnp.float32),
        scratch_shapes=[
            pltpu.VMEM((r, 1), jnp.float32),
            pltpu.VMEM((r, 1), jnp.float32),
            pltpu.VMEM((r, 1), jnp.float32),
        ],
    )(lab2, inp)


def _sc_partial(inp_flat, lab_i32, b, c):
    sc_rows = b - _TC_ROWS
    rpw = sc_rows // _NW
    nv = _CH // 16
    nch = c // _CH
    mesh = plsc.VectorSubcoreMesh(core_axis_name="c", subcore_axis_name="s")
    f32 = jnp.float32

    @functools.partial(
        pl.kernel,
        mesh=mesh,
        out_type=(
            jax.ShapeDtypeStruct((sc_rows, 16), f32),
            jax.ShapeDtypeStruct((sc_rows, 16), f32),
            jax.ShapeDtypeStruct((sc_rows, 16), f32),
        ),
        scratch_types=[
            pltpu.VMEM((_CH,), f32),
            pltpu.VMEM((rpw,), jnp.int32),
            pltpu.VMEM((16,), f32),
            pltpu.VMEM((16,), f32),
            pltpu.VMEM((16,), f32),
        ],
    )
    def _sck(x_hbm, lab_hbm, m_out, s_out, g_out, buf, labv, gvm, mv, sv):
        wid = lax.axis_index("s") * _NC + lax.axis_index("c")
        row0 = _TC_ROWS + wid * rpw
        pltpu.sync_copy(lab_hbm.at[pl.ds(row0, rpw)], labv)
        lane = lax.iota(jnp.int32, 16)
        labs_all = labv[...]

        def row_body(r):
            grow = row0 + r
            labs = labs_all[r]
            lab_chunk = jnp.int32(0)
            for t in range(1, nch):
                lab_chunk = lab_chunk + (labs >= t * _CH).astype(jnp.int32)
            lab_off = labs - lab_chunk * _CH
            vid = lax.shift_right_logical(lab_off, 4)
            pos = lab_off - vid * 16
            gvm[...] = jnp.zeros((16,), f32)

            def chunk_body(cidx, carry):
                m_l, s_l = carry
                pltpu.sync_copy(
                    x_hbm.at[pl.ds(grow * c + cidx * _CH, _CH)], buf)

                @pl.when(cidx == lab_chunk)
                def _fix():
                    start = pl.multiple_of(vid * 16, 16)
                    lvec = buf[pl.ds(start, 16)]
                    mask = lane == pos
                    gvm[...] = jnp.where(mask, lvec, 0.0)
                    buf[pl.ds(start, 16)] = jnp.where(mask, 0.0, lvec)

                def p1(i, wc):
                    res = wc
                    for u in range(_U):
                        off = pl.multiple_of((i * _U + u) * 16, 16)
                        res = jnp.maximum(res, buf[pl.ds(off, 16)])
                    return res

                w = lax.fori_loop(0, nv // _U, p1,
                                  jnp.full((16,), 0.25, f32))
                ub = w * w * 64.0 - 4.0
                m_new = jnp.maximum(m_l, ub)

                def p2(i, acc):
                    for u in range(_U):
                        off = pl.multiple_of((i * _U + u) * 16, 16)
                        v = buf[pl.ds(off, 16)]
                        l = jnp.where(v > -_M, v * v * 64.0 - 4.0, 0.0)
                        acc = acc + jnp.exp(l - m_new)
                    return acc

                s_new = s_l * jnp.exp(m_l - m_new) + lax.fori_loop(
                    0, nv // _U, p2, jnp.zeros((16,), f32))
                return (m_new, s_new)

            m_l, s_l = lax.fori_loop(
                0, nch, chunk_body,
                (jnp.zeros((16,), f32), jnp.zeros((16,), f32)))
            mv[...] = m_l
            sv[...] = s_l
            lrow = wid * rpw + r
            pltpu.sync_copy(mv, m_out.at[lrow])
            pltpu.sync_copy(sv, s_out.at[lrow])
            pltpu.sync_copy(gvm, g_out.at[lrow])

        for r in range(rpw):
            row_body(r)

    return _sck(inp_flat, lab_i32)


def _combine_kernel(tc_ref, m_ref, s_ref, g_ref, out_ref, *, n_rows):
    m = m_ref[...]
    m2 = jnp.max(m, axis=1, keepdims=True)
    stot = jnp.sum(s_ref[...] * jnp.exp(m - m2), axis=1, keepdims=True)
    g = jnp.sum(g_ref[...], axis=1, keepdims=True)
    spec = jnp.maximum((1.0 + _M) - g, 0.0) * (g - (1.0 - _M)) * 64.0
    scorr = stot - jnp.exp(-4.0 - m2) + jnp.exp(spec - m2)
    loss = m2 + jnp.log(scorr) - spec
    out_ref[0, 0] = (jnp.sum(loss) + tc_ref[0, 0]) / n_rows


def _combine(tc_part, m, s, g, b):
    shp = m.shape
    return pl.pallas_call(
        functools.partial(_combine_kernel, n_rows=b),
        in_specs=[
            pl.BlockSpec(memory_space=pltpu.SMEM),
            pl.BlockSpec(shp, lambda: (0, 0)),
            pl.BlockSpec(shp, lambda: (0, 0)),
            pl.BlockSpec(shp, lambda: (0, 0)),
        ],
        out_specs=pl.BlockSpec(memory_space=pltpu.SMEM),
        out_shape=jax.ShapeDtypeStruct((1, 1), jnp.float32),
    )(tc_part, m, s, g)


def kernel(inp, label):
    b, c = inp.shape
    lab_i32 = label.astype(jnp.int32)
    lab2 = lab_i32.reshape(b, 1)
    tc_part = _tc_partial(inp, lab2)
    m, s, g = _sc_partial(inp.reshape(-1), lab_i32, b, c)
    out = _combine(tc_part, m, s, g, b)
    return out[0, 0]


# speedup vs baseline: 2.2901x; 2.2901x over previous
import functools

import jax
import jax.numpy as jnp
from jax.experimental import pallas as pl
from jax.experimental.pallas import tpu as pltpu

_M = 0.25
_A = 64.0 * 1.4426950408889634
_LN2 = 0.6931471805599453
_NEG_INF = float("-inf")


def _loss_kernel(label_ref, x_ref, out_ref, m_ref, s_ref, g_ref, *, n_cols,
                 block_cols):
    k = pl.program_id(0)
    nk = pl.num_programs(0)

    @pl.when(k == 0)
    def _init():
        m_ref[...] = jnp.full(m_ref.shape, _NEG_INF, m_ref.dtype)
        s_ref[...] = jnp.zeros(s_ref.shape, s_ref.dtype)
        g_ref[...] = jnp.zeros(g_ref.shape, g_ref.dtype)

    def _accum(mask_invalid):
        x = x_ref[...]
        labloc = label_ref[...] - k * block_cols
        col = jax.lax.broadcasted_iota(jnp.int32, x.shape, 1)
        is_lab = col == labloc
        q = x * x * _A - (_A / 16.0)
        dense = jnp.where(x > -_M, q, 0.0)
        if mask_invalid:
            drop = is_lab | (col >= n_cols - k * block_cols)
        else:
            drop = is_lab
        l2 = jnp.where(drop, _NEG_INF, dense)
        g_ref[...] += jnp.sum(jnp.where(is_lab, x, 0.0), axis=1,
                              keepdims=True)
        bm = jnp.max(l2, axis=1, keepdims=True)
        m_old = m_ref[...]
        m_new = jnp.maximum(m_old, bm)
        s_ref[...] = s_ref[...] * jnp.exp2(m_old - m_new) + jnp.sum(
            jnp.exp2(l2 - m_new), axis=1, keepdims=True)
        m_ref[...] = m_new

    @pl.when(k < nk - 1)
    def _main():
        _accum(False)

    @pl.when(k == nk - 1)
    def _last():
        _accum(True)

        g = g_ref[...]
        spec2 = jnp.where(g < 1.0 + _M,
                          (2.0 * _A) * g - g * g * _A - 0.9375 * _A, 0.0)
        m2 = m_ref[...]
        s_true = s_ref[...] + jnp.exp2(spec2 - m2)
        loss = (m2 + jnp.log2(s_true) - spec2) * _LN2
        out_ref[0, 0] = jnp.sum(loss) / loss.shape[0]


def kernel(inp, label):
    b, c = inp.shape
    block_cols = 4096
    nk = pl.cdiv(c, block_cols)
    lab2 = label.astype(jnp.int32).reshape(b, 1)
    out = pl.pallas_call(
        functools.partial(_loss_kernel, n_cols=c, block_cols=block_cols),
        grid=(nk,),
        in_specs=[
            pl.BlockSpec((b, 1), lambda k: (0, 0)),
            pl.BlockSpec((b, block_cols), lambda k: (0, k)),
        ],
        out_specs=pl.BlockSpec(memory_space=pltpu.SMEM),
        out_shape=jax.ShapeDtypeStruct((1, 1), jnp.float32),
        scratch_shapes=[
            pltpu.VMEM((b, 1), jnp.float32),
            pltpu.VMEM((b, 1), jnp.float32),
            pltpu.VMEM((b, 1), jnp.float32),
        ],
    )(lab2, inp)
    return out[0, 0]
